# baseline (speedup 1.0000x reference)
import jax
import jax.numpy as jnp
from jax.experimental import pallas as pl
from jax.experimental.pallas import tpu as pltpu


def _pad_flat(x):
    N, C, H, W = x.shape
    xp = jnp.pad(x, ((0, 0), (0, 0), (1, 1), (1, 1)))
    return xp.reshape(N, C, (H + 2) * (W + 2)).astype(jnp.bfloat16)


def _ring_mask(H, W):
    m = jnp.zeros((H + 2, W + 2), jnp.float32).at[1:H + 1, 1:W + 1].set(1.0)
    return m.reshape(1, (H + 2) * (W + 2))


def _dense_block(parts, mask, layers, *, Wp, first_conv=None,
                 emit_passthrough=True, final_mm=None):
    N = parts[0].shape[0]
    Pp = parts[0].shape[2]
    Cs = [p.shape[1] for p in parts]
    nparts = len(parts)
    L = len(layers)
    g = layers[0][2].shape[0] // 9
    C0 = first_conv[0].shape[0] if first_conv is not None else sum(Cs)
    Ctot = C0 + L * g
    Cfeat = Ctot if final_mm is not None else C0 + (L - 1) * g
    Cmaxh = C0 + (L - 1) * g
    E = Wp + 1
    Pe = Pp + 2 * E
    offs = tuple(dy * Wp + dx for dy in range(3) for dx in range(3))

    def body(*refs):
        i = 0
        prefs = refs[:nparts]
        i = nparts
        m_ref = refs[i]
        i += 1
        if first_conv is not None:
            fcw_ref, fcb_ref = refs[i], refs[i + 1]
            i += 2
        lrefs = refs[i:i + 4 * L]
        i += 4 * L
        if final_mm is not None:
            lw_ref, lb_ref = refs[i], refs[i + 1]
            i += 2
        out_ref = refs[i]
        i += 1
        feat, hbuf = refs[i], refs[i + 1]
        if first_conv is not None:
            htaps = refs[i + 2]

        m = m_ref[...]
        hbuf[:, :E] = jnp.zeros((Cmaxh, E), jnp.bfloat16)
        hbuf[:, E + Pp:] = jnp.zeros((Cmaxh, E), jnp.bfloat16)

        if first_conv is not None:
            cin0 = Cs[0]
            hbuf[:cin0, E:E + Pp] = prefs[0][0]
            for t in range(9):
                htaps[cin0 * t:cin0 * (t + 1), :] = \
                    hbuf[:cin0, offs[t]:offs[t] + Pp]
            y0 = jnp.dot(fcw_ref[...], htaps[...],
                         preferred_element_type=jnp.float32)
            f0 = ((y0 + fcb_ref[...]) * m).astype(jnp.bfloat16)
            feat[:C0, :] = f0
            if emit_passthrough and final_mm is None:
                out_ref[0, :C0, :] = f0
        else:
            base = 0
            for pi in range(nparts):
                c = Cs[pi]
                xpart = prefs[pi][0]
                feat[base:base + c, :] = xpart
                if emit_passthrough and final_mm is None:
                    out_ref[0, base:base + c, :] = xpart
                base += c

        for l in range(L):
            cin = C0 + l * g
            s_ref, t_ref, w_ref, b_ref = lrefs[4 * l:4 * l + 4]
            f = feat[:cin, :].astype(jnp.float32)
            h = (jnp.maximum(f * s_ref[...] + t_ref[...], 0.0)
                 * m).astype(jnp.bfloat16)
            hbuf[:cin, E:E + Pp] = h
            y = jnp.dot(w_ref[...], hbuf[:cin, :],
                        preferred_element_type=jnp.float32)
            acc = y[0:g, offs[0]:offs[0] + Pp]
            for t in range(1, 9):
                acc = acc + y[g * t:g * (t + 1), offs[t]:offs[t] + Pp]
            z = ((acc + b_ref[...]) * m).astype(jnp.bfloat16)
            if final_mm is not None or l + 1 < L:
                feat[C0 + l * g:C0 + (l + 1) * g, :] = z
            if final_mm is None:
                obase = (C0 + l * g) if emit_passthrough else (l * g)
                out_ref[0, obase:obase + g, :] = z

        if final_mm is not None:
            yf = jnp.dot(lw_ref[...], feat[...],
                         preferred_element_type=jnp.float32)
            out_ref[0] = yf + lb_ref[...]

    in_specs = []
    args = []
    for p in parts:
        c = p.shape[1]
        in_specs.append(pl.BlockSpec((1, c, Pp), lambda n: (n, 0, 0)))
        args.append(p)
    in_specs.append(pl.BlockSpec((1, Pp), lambda n: (0, 0)))
    args.append(mask)
    if first_conv is not None:
        fcw, fcb = first_conv
        in_specs += [pl.BlockSpec(fcw.shape, lambda n: (0, 0)),
                     pl.BlockSpec(fcb.shape, lambda n: (0, 0))]
        args += [fcw, fcb]
    for (s, t, w, b) in layers:
        in_specs += [pl.BlockSpec(s.shape, lambda n: (0, 0)),
                     pl.BlockSpec(t.shape, lambda n: (0, 0)),
                     pl.BlockSpec(w.shape, lambda n: (0, 0)),
                     pl.BlockSpec(b.shape, lambda n: (0, 0))]
        args += [s, t, w, b]
    if final_mm is not None:
        lw, lb = final_mm
        in_specs += [pl.BlockSpec(lw.shape, lambda n: (0, 0)),
                     pl.BlockSpec(lb.shape, lambda n: (0, 0))]
        args += [lw, lb]

    if final_mm is not None:
        Cout, odtype = 1, jnp.float32
    elif emit_passthrough:
        Cout, odtype = Ctot, jnp.bfloat16
    else:
        Cout, odtype = L * g, jnp.bfloat16

    scratch = [pltpu.VMEM((Cfeat, Pp), jnp.bfloat16),
               pltpu.VMEM((Cmaxh, Pe), jnp.bfloat16)]
    if first_conv is not None:
        scratch.append(pltpu.VMEM((9 * Cs[0], Pp), jnp.bfloat16))

    return pl.pallas_call(
        body,
        out_shape=jax.ShapeDtypeStruct((N, Cout, Pp), odtype),
        grid=(N,),
        in_specs=in_specs,
        out_specs=pl.BlockSpec((1, Cout, Pp), lambda n: (n, 0, 0)),
        scratch_shapes=scratch,
        compiler_params=pltpu.CompilerParams(dimension_semantics=("parallel",)),
    )(*args)


def _td_call(corners, scale, shift, w, b, Mo):
    N, C, M4 = corners.shape

    def body(x_ref, s_ref, t_ref, w_ref, b_ref, o_ref):
        h = (jnp.maximum(x_ref[0].astype(jnp.float32) * s_ref[...]
                         + t_ref[...], 0.0)).astype(jnp.bfloat16)
        d = jnp.dot(w_ref[...], h, preferred_element_type=jnp.float32)
        r = jnp.maximum(jnp.maximum(d[:, 0 * Mo:1 * Mo], d[:, 1 * Mo:2 * Mo]),
                        jnp.maximum(d[:, 2 * Mo:3 * Mo], d[:, 3 * Mo:4 * Mo]))
        o_ref[0] = (r + b_ref[...]).astype(jnp.bfloat16)

    return pl.pallas_call(
        body,
        out_shape=jax.ShapeDtypeStruct((N, C, Mo), jnp.bfloat16),
        grid=(N,),
        in_specs=[pl.BlockSpec((1, C, M4), lambda n: (n, 0, 0)),
                  pl.BlockSpec((C, 1), lambda n: (0, 0)),
                  pl.BlockSpec((C, 1), lambda n: (0, 0)),
                  pl.BlockSpec((C, C), lambda n: (0, 0)),
                  pl.BlockSpec((C, 1), lambda n: (0, 0))],
        out_specs=pl.BlockSpec((1, C, Mo), lambda n: (n, 0, 0)),
        compiler_params=pltpu.CompilerParams(dimension_semantics=("parallel",)),
    )(corners, scale, shift, w, b)


def _transition_down(xparts, scale, shift, w, b, H, W):
    N = xparts[0].shape[0]
    Hp, Wp, Ho, Wo = H + 2, W + 2, H // 2, W // 2
    cs = []
    for xp in xparts:
        C = xp.shape[1]
        x4 = xp.reshape(N, C, Hp, Wp)
        cs.append(jnp.stack([x4[:, :, 1 + r:1 + 2 * Ho:2, 1 + c:1 + 2 * Wo:2]
                             for r in range(2) for c in range(2)], axis=2))
    corners = jnp.concatenate(cs, axis=1) if len(cs) > 1 else cs[0]
    corners = corners.reshape(N, corners.shape[1], 4 * Ho * Wo)
    out = _td_call(corners, scale, shift, w, b, Ho * Wo)
    return _pad_flat(out.reshape(N, out.shape[1], Ho, Wo)), Ho, Wo


def _mm_call(x, w, b):
    N, K, M = x.shape
    Cout = w.shape[0]

    def body(x_ref, w_ref, b_ref, o_ref):
        o_ref[0] = (jnp.dot(w_ref[...], x_ref[0],
                            preferred_element_type=jnp.float32)
                    + b_ref[...]).astype(jnp.bfloat16)

    return pl.pallas_call(
        body,
        out_shape=jax.ShapeDtypeStruct((N, Cout, M), jnp.bfloat16),
        grid=(N,),
        in_specs=[pl.BlockSpec((1, K, M), lambda n: (n, 0, 0)),
                  pl.BlockSpec((Cout, K), lambda n: (0, 0)),
                  pl.BlockSpec((Cout, 1), lambda n: (0, 0))],
        out_specs=pl.BlockSpec((1, Cout, M), lambda n: (n, 0, 0)),
        compiler_params=pltpu.CompilerParams(dimension_semantics=("parallel",)),
    )(x, w, b)


def _transition_up(x, w, b, H, W, Hs, Ws):
    N, Cin, _ = x.shape
    Hp, Wp = H + 2, W + 2
    x4 = x.reshape(N, Cin, Hp, Wp)
    taps = jnp.concatenate([x4[:, :, r:r + H, c:c + W]
                            for r in range(2) for c in range(2)], axis=1)
    taps = taps.reshape(N, 4 * Cin, H * W)
    out = _mm_call(taps, w, b)
    Cout = w.shape[0] // 4
    out = out.reshape(N, 2, 2, Cout, H, W)
    out = jnp.transpose(out, (0, 3, 4, 1, 5, 2)).reshape(N, Cout, 2 * H, 2 * W)
    off_h = (2 * H + 1 - Hs) // 2
    off_w = (2 * W + 1 - Ws) // 2
    out = out[:, :, off_h:off_h + Hs, off_w:off_w + Ws]
    return _pad_flat(out)


def kernel(x, first_conv_w, first_conv_b, down0_0_scale, down0_0_shift,
           down0_0_w, down0_0_b, down0_1_scale, down0_1_shift, down0_1_w,
           down0_1_b, down1_0_scale, down1_0_shift, down1_0_w, down1_0_b,
           down1_1_scale, down1_1_shift, down1_1_w, down1_1_b, td0_scale,
           td0_shift, td0_w, td0_b, td1_scale, td1_shift, td1_w, td1_b,
           bn0_scale, bn0_shift, bn0_w, bn0_b, bn1_scale, bn1_shift, bn1_w,
           bn1_b, tu0_w, tu0_b, tu1_w, tu1_b, up0_0_scale, up0_0_shift,
           up0_0_w, up0_0_b, up0_1_scale, up0_1_shift, up0_1_w, up0_1_b,
           up1_0_scale, up1_0_shift, up1_0_w, up1_0_b, up1_1_scale,
           up1_1_shift, up1_1_w, up1_1_b, last_conv_w, last_conv_b):
    N, _, H, W = x.shape

    stk = lambda w: w.reshape(w.shape[0] * w.shape[1], w.shape[2])
    C1 = first_conv_w.shape[1]
    fcw27 = jnp.transpose(first_conv_w, (1, 0, 2)).reshape(C1, 27)

    xp = _pad_flat(x)
    m0 = _ring_mask(H, W)

    a = _dense_block(
        [xp], m0,
        [(down0_0_scale, down0_0_shift, stk(down0_0_w), down0_0_b),
         (down0_1_scale, down0_1_shift, stk(down0_1_w), down0_1_b)],
        Wp=W + 2, first_conv=(fcw27, first_conv_b),
        emit_passthrough=True)

    b1, H1, W1 = _transition_down([a], td0_scale, td0_shift, td0_w, td0_b,
                                  H, W)
    m1 = _ring_mask(H1, W1)

    c_new = _dense_block(
        [b1], m1,
        [(down1_0_scale, down1_0_shift, stk(down1_0_w), down1_0_b),
         (down1_1_scale, down1_1_shift, stk(down1_1_w), down1_1_b)],
        Wp=W1 + 2, emit_passthrough=False)

    d, H2, W2 = _transition_down([b1, c_new], td1_scale, td1_shift,
                                 td1_w, td1_b, H1, W1)
    m2 = _ring_mask(H2, W2)

    e = _dense_block(
        [d], m2,
        [(bn0_scale, bn0_shift, stk(bn0_w), bn0_b),
         (bn1_scale, bn1_shift, stk(bn1_w), bn1_b)],
        Wp=W2 + 2, emit_passthrough=False)

    u0 = _transition_up(e, tu0_w, tu0_b, H2, W2, H1, W1)
    f = _dense_block(
        [u0, b1, c_new], m1,
        [(up0_0_scale, up0_0_shift, stk(up0_0_w), up0_0_b),
         (up0_1_scale, up0_1_shift, stk(up0_1_w), up0_1_b)],
        Wp=W1 + 2, emit_passthrough=False)

    u1 = _transition_up(f, tu1_w, tu1_b, H1, W1, H, W)
    out = _dense_block(
        [u1, a], m0,
        [(up1_0_scale, up1_0_shift, stk(up1_0_w), up1_0_b),
         (up1_1_scale, up1_1_shift, stk(up1_1_w), up1_1_b)],
        Wp=W + 2, final_mm=(last_conv_w, last_conv_b))

    out = out.reshape(N, 1, H + 2, W + 2)[:, :, 1:H + 1, 1:W + 1]
    return out


# speedup vs baseline: 1.0169x; 1.0169x over previous
import jax
import jax.numpy as jnp
from jax.experimental import pallas as pl
from jax.experimental.pallas import tpu as pltpu


def _pad_flat(x):
    N, C, H, W = x.shape
    xp = jnp.pad(x, ((0, 0), (0, 0), (1, 1), (1, 1)))
    return xp.reshape(N, C, (H + 2) * (W + 2)).astype(jnp.bfloat16)


def _ring_mask(H, W):
    m = jnp.zeros((H + 2, W + 2), jnp.float32).at[1:H + 1, 1:W + 1].set(1.0)
    return m.reshape(1, (H + 2) * (W + 2))


def _dense_block(parts, mask, layers, *, Wp, first_conv=None,
                 emit_passthrough=True, final_mm=None):
    N = parts[0].shape[0]
    Pp = parts[0].shape[2]
    Cs = [p.shape[1] for p in parts]
    nparts = len(parts)
    L = len(layers)
    g = layers[0][2].shape[1] // 3
    C0 = first_conv[0].shape[0] if first_conv is not None else sum(Cs)
    Ctot = C0 + L * g
    Cfeat = Ctot if final_mm is not None else C0 + (L - 1) * g
    Cmaxh = C0 + (L - 1) * g
    E = Wp + 1
    Pe = Pp + 2 * E
    offs = tuple(dy * Wp + dx for dy in range(3) for dx in range(3))

    def body(*refs):
        i = 0
        prefs = refs[:nparts]
        i = nparts
        m_ref = refs[i]
        i += 1
        if first_conv is not None:
            fcw_ref, fcb_ref = refs[i], refs[i + 1]
            i += 2
        lrefs = refs[i:i + 4 * L]
        i += 4 * L
        if final_mm is not None:
            lw_ref, lb_ref = refs[i], refs[i + 1]
            i += 2
        out_ref = refs[i]
        i += 1
        feat, hbuf = refs[i], refs[i + 1]
        if first_conv is not None:
            htaps = refs[i + 2]

        m = m_ref[...]
        hbuf[:, :E] = jnp.zeros((Cmaxh, E), jnp.bfloat16)
        hbuf[:, E + Pp:] = jnp.zeros((Cmaxh, E), jnp.bfloat16)

        if first_conv is not None:
            cin0 = Cs[0]
            hbuf[:cin0, E:E + Pp] = prefs[0][0]
            for t in range(9):
                htaps[cin0 * t:cin0 * (t + 1), :] = \
                    hbuf[:cin0, offs[t]:offs[t] + Pp]
            y0 = jnp.dot(fcw_ref[...], htaps[...],
                         preferred_element_type=jnp.float32)
            f0 = ((y0 + fcb_ref[...]) * m).astype(jnp.bfloat16)
            feat[:C0, :] = f0
            if emit_passthrough and final_mm is None:
                out_ref[0, :C0, :] = f0
        else:
            base = 0
            for pi in range(nparts):
                c = Cs[pi]
                xpart = prefs[pi][0]
                feat[base:base + c, :] = xpart
                if emit_passthrough and final_mm is None:
                    out_ref[0, base:base + c, :] = xpart
                base += c

        for l in range(L):
            cin = C0 + l * g
            s_ref, t_ref, w_ref, b_ref = lrefs[4 * l:4 * l + 4]
            f = feat[:cin, :].astype(jnp.float32)
            h = (jnp.maximum(f * s_ref[...] + t_ref[...], 0.0)
                 * m).astype(jnp.bfloat16)
            hbuf[:cin, E:E + Pp] = h
            Wy = Pp + 2 * Wp
            ysum = None
            for dx in range(3):
                yd = jnp.dot(w_ref[dx], hbuf[:cin, dx:dx + Wy],
                             preferred_element_type=jnp.float32)
                ysum = yd if ysum is None else ysum + yd
            acc = ysum[0:g, 0:Pp]
            for dy in range(1, 3):
                acc = acc + ysum[g * dy:g * (dy + 1),
                                 dy * Wp:dy * Wp + Pp]
            z = ((acc + b_ref[...]) * m).astype(jnp.bfloat16)
            if final_mm is not None or l + 1 < L:
                feat[C0 + l * g:C0 + (l + 1) * g, :] = z
            if final_mm is None:
                obase = (C0 + l * g) if emit_passthrough else (l * g)
                out_ref[0, obase:obase + g, :] = z

        if final_mm is not None:
            yf = jnp.dot(lw_ref[...], feat[...],
                         preferred_element_type=jnp.float32)
            out_ref[0] = yf + lb_ref[...]

    in_specs = []
    args = []
    for p in parts:
        c = p.shape[1]
        in_specs.append(pl.BlockSpec((1, c, Pp), lambda n: (n, 0, 0)))
        args.append(p)
    in_specs.append(pl.BlockSpec((1, Pp), lambda n: (0, 0)))
    args.append(mask)
    if first_conv is not None:
        fcw, fcb = first_conv
        in_specs += [pl.BlockSpec(fcw.shape, lambda n: (0, 0)),
                     pl.BlockSpec(fcb.shape, lambda n: (0, 0))]
        args += [fcw, fcb]
    for (s, t, w, b) in layers:
        in_specs += [pl.BlockSpec(s.shape, lambda n: (0, 0)),
                     pl.BlockSpec(t.shape, lambda n: (0, 0)),
                     pl.BlockSpec(w.shape, lambda n: (0, 0, 0)),
                     pl.BlockSpec(b.shape, lambda n: (0, 0))]
        args += [s, t, w, b]
    if final_mm is not None:
        lw, lb = final_mm
        in_specs += [pl.BlockSpec(lw.shape, lambda n: (0, 0)),
                     pl.BlockSpec(lb.shape, lambda n: (0, 0))]
        args += [lw, lb]

    if final_mm is not None:
        Cout, odtype = 1, jnp.float32
    elif emit_passthrough:
        Cout, odtype = Ctot, jnp.bfloat16
    else:
        Cout, odtype = L * g, jnp.bfloat16

    scratch = [pltpu.VMEM((Cfeat, Pp), jnp.bfloat16),
               pltpu.VMEM((Cmaxh, Pe), jnp.bfloat16)]
    if first_conv is not None:
        scratch.append(pltpu.VMEM((9 * Cs[0], Pp), jnp.bfloat16))

    return pl.pallas_call(
        body,
        out_shape=jax.ShapeDtypeStruct((N, Cout, Pp), odtype),
        grid=(N,),
        in_specs=in_specs,
        out_specs=pl.BlockSpec((1, Cout, Pp), lambda n: (n, 0, 0)),
        scratch_shapes=scratch,
        compiler_params=pltpu.CompilerParams(dimension_semantics=("parallel",)),
    )(*args)


def _td_call(corners, scale, shift, w, b, Mo):
    N, C, M4 = corners.shape

    def body(x_ref, s_ref, t_ref, w_ref, b_ref, o_ref):
        h = (jnp.maximum(x_ref[0].astype(jnp.float32) * s_ref[...]
                         + t_ref[...], 0.0)).astype(jnp.bfloat16)
        d = jnp.dot(w_ref[...], h, preferred_element_type=jnp.float32)
        r = jnp.maximum(jnp.maximum(d[:, 0 * Mo:1 * Mo], d[:, 1 * Mo:2 * Mo]),
                        jnp.maximum(d[:, 2 * Mo:3 * Mo], d[:, 3 * Mo:4 * Mo]))
        o_ref[0] = (r + b_ref[...]).astype(jnp.bfloat16)

    return pl.pallas_call(
        body,
        out_shape=jax.ShapeDtypeStruct((N, C, Mo), jnp.bfloat16),
        grid=(N,),
        in_specs=[pl.BlockSpec((1, C, M4), lambda n: (n, 0, 0)),
                  pl.BlockSpec((C, 1), lambda n: (0, 0)),
                  pl.BlockSpec((C, 1), lambda n: (0, 0)),
                  pl.BlockSpec((C, C), lambda n: (0, 0)),
                  pl.BlockSpec((C, 1), lambda n: (0, 0))],
        out_specs=pl.BlockSpec((1, C, Mo), lambda n: (n, 0, 0)),
        compiler_params=pltpu.CompilerParams(dimension_semantics=("parallel",)),
    )(corners, scale, shift, w, b)


def _transition_down(xparts, scale, shift, w, b, H, W):
    N = xparts[0].shape[0]
    Hp, Wp, Ho, Wo = H + 2, W + 2, H // 2, W // 2
    cs = []
    for xp in xparts:
        C = xp.shape[1]
        x4 = xp.reshape(N, C, Hp, Wp)
        cs.append(jnp.stack([x4[:, :, 1 + r:1 + 2 * Ho:2, 1 + c:1 + 2 * Wo:2]
                             for r in range(2) for c in range(2)], axis=2))
    corners = jnp.concatenate(cs, axis=1) if len(cs) > 1 else cs[0]
    corners = corners.reshape(N, corners.shape[1], 4 * Ho * Wo)
    out = _td_call(corners, scale, shift, w, b, Ho * Wo)
    return _pad_flat(out.reshape(N, out.shape[1], Ho, Wo)), Ho, Wo


def _mm_call(x, w, b):
    N, K, M = x.shape
    Cout = w.shape[0]

    def body(x_ref, w_ref, b_ref, o_ref):
        o_ref[0] = (jnp.dot(w_ref[...], x_ref[0],
                            preferred_element_type=jnp.float32)
                    + b_ref[...]).astype(jnp.bfloat16)

    return pl.pallas_call(
        body,
        out_shape=jax.ShapeDtypeStruct((N, Cout, M), jnp.bfloat16),
        grid=(N,),
        in_specs=[pl.BlockSpec((1, K, M), lambda n: (n, 0, 0)),
                  pl.BlockSpec((Cout, K), lambda n: (0, 0)),
                  pl.BlockSpec((Cout, 1), lambda n: (0, 0))],
        out_specs=pl.BlockSpec((1, Cout, M), lambda n: (n, 0, 0)),
        compiler_params=pltpu.CompilerParams(dimension_semantics=("parallel",)),
    )(x, w, b)


def _transition_up(x, w, b, H, W, Hs, Ws):
    N, Cin, _ = x.shape
    Hp, Wp = H + 2, W + 2
    x4 = x.reshape(N, Cin, Hp, Wp)
    taps = jnp.concatenate([x4[:, :, r:r + H, c:c + W]
                            for r in range(2) for c in range(2)], axis=1)
    taps = taps.reshape(N, 4 * Cin, H * W)
    out = _mm_call(taps, w, b)
    Cout = w.shape[0] // 4
    out = out.reshape(N, 2, 2, Cout, H, W)
    out = jnp.transpose(out, (0, 3, 4, 1, 5, 2)).reshape(N, Cout, 2 * H, 2 * W)
    off_h = (2 * H + 1 - Hs) // 2
    off_w = (2 * W + 1 - Ws) // 2
    out = out[:, :, off_h:off_h + Hs, off_w:off_w + Ws]
    return _pad_flat(out)


def kernel(x, first_conv_w, first_conv_b, down0_0_scale, down0_0_shift,
           down0_0_w, down0_0_b, down0_1_scale, down0_1_shift, down0_1_w,
           down0_1_b, down1_0_scale, down1_0_shift, down1_0_w, down1_0_b,
           down1_1_scale, down1_1_shift, down1_1_w, down1_1_b, td0_scale,
           td0_shift, td0_w, td0_b, td1_scale, td1_shift, td1_w, td1_b,
           bn0_scale, bn0_shift, bn0_w, bn0_b, bn1_scale, bn1_shift, bn1_w,
           bn1_b, tu0_w, tu0_b, tu1_w, tu1_b, up0_0_scale, up0_0_shift,
           up0_0_w, up0_0_b, up0_1_scale, up0_1_shift, up0_1_w, up0_1_b,
           up1_0_scale, up1_0_shift, up1_0_w, up1_0_b, up1_1_scale,
           up1_1_shift, up1_1_w, up1_1_b, last_conv_w, last_conv_b):
    N, _, H, W = x.shape

    def stk(w):
        g, cin = w.shape[1], w.shape[2]
        return jnp.transpose(w.reshape(3, 3, g, cin),
                             (1, 0, 2, 3)).reshape(3, 3 * g, cin)
    C1 = first_conv_w.shape[1]
    fcw27 = jnp.transpose(first_conv_w, (1, 0, 2)).reshape(C1, 27)

    xp = _pad_flat(x)
    m0 = _ring_mask(H, W)

    a = _dense_block(
        [xp], m0,
        [(down0_0_scale, down0_0_shift, stk(down0_0_w), down0_0_b),
         (down0_1_scale, down0_1_shift, stk(down0_1_w), down0_1_b)],
        Wp=W + 2, first_conv=(fcw27, first_conv_b),
        emit_passthrough=True)

    b1, H1, W1 = _transition_down([a], td0_scale, td0_shift, td0_w, td0_b,
                                  H, W)
    m1 = _ring_mask(H1, W1)

    c_new = _dense_block(
        [b1], m1,
        [(down1_0_scale, down1_0_shift, stk(down1_0_w), down1_0_b),
         (down1_1_scale, down1_1_shift, stk(down1_1_w), down1_1_b)],
        Wp=W1 + 2, emit_passthrough=False)

    d, H2, W2 = _transition_down([b1, c_new], td1_scale, td1_shift,
                                 td1_w, td1_b, H1, W1)
    m2 = _ring_mask(H2, W2)

    e = _dense_block(
        [d], m2,
        [(bn0_scale, bn0_shift, stk(bn0_w), bn0_b),
         (bn1_scale, bn1_shift, stk(bn1_w), bn1_b)],
        Wp=W2 + 2, emit_passthrough=False)

    u0 = _transition_up(e, tu0_w, tu0_b, H2, W2, H1, W1)
    f = _dense_block(
        [u0, b1, c_new], m1,
        [(up0_0_scale, up0_0_shift, stk(up0_0_w), up0_0_b),
         (up0_1_scale, up0_1_shift, stk(up0_1_w), up0_1_b)],
        Wp=W1 + 2, emit_passthrough=False)

    u1 = _transition_up(f, tu1_w, tu1_b, H1, W1, H, W)
    out = _dense_block(
        [u1, a], m0,
        [(up1_0_scale, up1_0_shift, stk(up1_0_w), up1_0_b),
         (up1_1_scale, up1_1_shift, stk(up1_1_w), up1_1_b)],
        Wp=W + 2, final_mm=(last_conv_w, last_conv_b))

    out = out.reshape(N, 1, H + 2, W + 2)[:, :, 1:H + 1, 1:W + 1]
    return out


# speedup vs baseline: 2.5227x; 2.4809x over previous
import jax
import jax.numpy as jnp
from jax.experimental import pallas as pl
from jax.experimental.pallas import tpu as pltpu


def _pad_flat(x):
    N, C, H, W = x.shape
    xp = jnp.pad(x, ((0, 0), (0, 0), (1, 1), (1, 1)))
    return xp.reshape(N, C, (H + 2) * (W + 2)).astype(jnp.bfloat16)


def _ring_mask(H, W):
    m = jnp.zeros((H + 2, W + 2), jnp.float32).at[1:H + 1, 1:W + 1].set(1.0)
    return m.reshape(1, (H + 2) * (W + 2))


def _dense_block(parts, mask, layers, *, Wp, first_conv=None,
                 emit_passthrough=True, final_mm=None, td=None):
    N = parts[0].shape[0]
    Pp = parts[0].shape[2]
    Cs = [p.shape[1] for p in parts]
    nparts = len(parts)
    L = len(layers)
    g = layers[0][2].shape[1] // 3
    C0 = first_conv[0].shape[0] if first_conv is not None else sum(Cs)
    Ctot = C0 + L * g
    Cfeat = Ctot if (final_mm is not None or td is not None) \
        else C0 + (L - 1) * g
    Cmaxh = C0 + (L - 1) * g
    E = Wp + 1
    Pe = Pp + 2 * E
    offs = tuple(dy * Wp + dx for dy in range(3) for dx in range(3))

    def body(*refs):
        i = 0
        prefs = refs[:nparts]
        i = nparts
        m_ref = refs[i]
        i += 1
        if first_conv is not None:
            fcw_ref, fcb_ref = refs[i], refs[i + 1]
            i += 2
        lrefs = refs[i:i + 4 * L]
        i += 4 * L
        if final_mm is not None:
            lw_ref, lb_ref = refs[i], refs[i + 1]
            i += 2
        if td is not None:
            tds_ref, tdt_ref, tdw_ref, tdb_ref = refs[i:i + 4]
            i += 4
        out_ref = refs[i]
        i += 1
        if td is not None:
            td_ref = refs[i]
            i += 1
        feat, hbuf = refs[i], refs[i + 1]
        if first_conv is not None:
            htaps = refs[i + 2]

        m = m_ref[...]
        hbuf[:, :E] = jnp.zeros((Cmaxh, E), jnp.bfloat16)
        hbuf[:, E + Pp:] = jnp.zeros((Cmaxh, E), jnp.bfloat16)

        if first_conv is not None:
            cin0 = Cs[0]
            hbuf[:cin0, E:E + Pp] = prefs[0][0]
            for t in range(9):
                htaps[cin0 * t:cin0 * (t + 1), :] = \
                    hbuf[:cin0, offs[t]:offs[t] + Pp]
            y0 = jnp.dot(fcw_ref[...], htaps[...],
                         preferred_element_type=jnp.float32)
            f0 = ((y0 + fcb_ref[...]) * m).astype(jnp.bfloat16)
            feat[:C0, :] = f0
            if emit_passthrough and final_mm is None:
                out_ref[0, :C0, :] = f0
        else:
            base = 0
            for pi in range(nparts):
                c = Cs[pi]
                xpart = prefs[pi][0]
                feat[base:base + c, :] = xpart
                if emit_passthrough and final_mm is None:
                    out_ref[0, base:base + c, :] = xpart
                base += c

        for l in range(L):
            cin = C0 + l * g
            s_ref, t_ref, w_ref, b_ref = lrefs[4 * l:4 * l + 4]
            f = feat[:cin, :].astype(jnp.float32)
            h = (jnp.maximum(f * s_ref[...] + t_ref[...], 0.0)
                 * m).astype(jnp.bfloat16)
            hbuf[:cin, E:E + Pp] = h
            Wy = Pp + 2 * Wp
            ysum = None
            for dx in range(3):
                yd = jnp.dot(w_ref[dx], hbuf[:cin, dx:dx + Wy],
                             preferred_element_type=jnp.float32)
                ysum = yd if ysum is None else ysum + yd
            acc = ysum[0:g, 0:Pp]
            for dy in range(1, 3):
                acc = acc + ysum[g * dy:g * (dy + 1),
                                 dy * Wp:dy * Wp + Pp]
            z = ((acc + b_ref[...]) * m).astype(jnp.bfloat16)
            if final_mm is not None or td is not None or l + 1 < L:
                feat[C0 + l * g:C0 + (l + 1) * g, :] = z
            if final_mm is None:
                obase = (C0 + l * g) if emit_passthrough else (l * g)
                out_ref[0, obase:obase + g, :] = z

        if final_mm is not None:
            yf = jnp.dot(lw_ref[...], feat[...],
                         preferred_element_type=jnp.float32)
            out_ref[0] = yf + lb_ref[...]

        if td is not None:
            ftd = feat[...].astype(jnp.float32)
            htd = (jnp.maximum(ftd * tds_ref[...] + tdt_ref[...], 0.0)
                   ).astype(jnp.bfloat16)
            dtd = jnp.dot(tdw_ref[...], htd,
                          preferred_element_type=jnp.float32)
            td_ref[0] = (dtd + tdb_ref[...]).astype(jnp.bfloat16)

    in_specs = []
    args = []
    for p in parts:
        c = p.shape[1]
        in_specs.append(pl.BlockSpec((1, c, Pp), lambda n: (n, 0, 0)))
        args.append(p)
    in_specs.append(pl.BlockSpec((1, Pp), lambda n: (0, 0)))
    args.append(mask)
    if first_conv is not None:
        fcw, fcb = first_conv
        in_specs += [pl.BlockSpec(fcw.shape, lambda n: (0, 0)),
                     pl.BlockSpec(fcb.shape, lambda n: (0, 0))]
        args += [fcw, fcb]
    for (s, t, w, b) in layers:
        in_specs += [pl.BlockSpec(s.shape, lambda n: (0, 0)),
                     pl.BlockSpec(t.shape, lambda n: (0, 0)),
                     pl.BlockSpec(w.shape, lambda n: (0, 0, 0)),
                     pl.BlockSpec(b.shape, lambda n: (0, 0))]
        args += [s, t, w, b]
    if final_mm is not None:
        lw, lb = final_mm
        in_specs += [pl.BlockSpec(lw.shape, lambda n: (0, 0)),
                     pl.BlockSpec(lb.shape, lambda n: (0, 0))]
        args += [lw, lb]
    if td is not None:
        for t in td:
            in_specs.append(pl.BlockSpec(t.shape, lambda n: (0, 0)))
        args += list(td)

    if final_mm is not None:
        Cout, odtype = 1, jnp.float32
    elif emit_passthrough:
        Cout, odtype = Ctot, jnp.bfloat16
    else:
        Cout, odtype = L * g, jnp.bfloat16

    out_shape = [jax.ShapeDtypeStruct((N, Cout, Pp), odtype)]
    out_specs = [pl.BlockSpec((1, Cout, Pp), lambda n: (n, 0, 0))]
    if td is not None:
        Ctd = td[2].shape[0]
        out_shape.append(jax.ShapeDtypeStruct((N, Ctd, Pp), jnp.bfloat16))
        out_specs.append(pl.BlockSpec((1, Ctd, Pp), lambda n: (n, 0, 0)))

    scratch = [pltpu.VMEM((Cfeat, Pp), jnp.bfloat16),
               pltpu.VMEM((Cmaxh, Pe), jnp.bfloat16)]
    if first_conv is not None:
        scratch.append(pltpu.VMEM((9 * Cs[0], Pp), jnp.bfloat16))

    res = pl.pallas_call(
        body,
        out_shape=out_shape,
        grid=(N,),
        in_specs=in_specs,
        out_specs=out_specs,
        scratch_shapes=scratch,
        compiler_params=pltpu.CompilerParams(dimension_semantics=("parallel",)),
    )(*args)
    return res if td is not None else res[0]


def _pool2(d, H, W):
    N, C, _ = d.shape
    d4 = d.reshape(N, C, H + 2, W + 2)[:, :, 1:H + 1, 1:W + 1]
    p = jax.lax.reduce_window(d4, jnp.array(-jnp.inf, jnp.bfloat16),
                              jax.lax.max, (1, 1, 2, 2), (1, 1, 2, 2),
                              "VALID")
    return _pad_flat(p), H // 2, W // 2


def _mm_call(x, w, b):
    N, K, M = x.shape
    Cout = w.shape[0]

    def body(x_ref, w_ref, b_ref, o_ref):
        o_ref[0] = (jnp.dot(w_ref[...], x_ref[0],
                            preferred_element_type=jnp.float32)
                    + b_ref[...]).astype(jnp.bfloat16)

    return pl.pallas_call(
        body,
        out_shape=jax.ShapeDtypeStruct((N, Cout, M), jnp.bfloat16),
        grid=(N,),
        in_specs=[pl.BlockSpec((1, K, M), lambda n: (n, 0, 0)),
                  pl.BlockSpec((Cout, K), lambda n: (0, 0)),
                  pl.BlockSpec((Cout, 1), lambda n: (0, 0))],
        out_specs=pl.BlockSpec((1, Cout, M), lambda n: (n, 0, 0)),
        compiler_params=pltpu.CompilerParams(dimension_semantics=("parallel",)),
    )(x, w, b)


def _transition_up(x, w, b, H, W, Hs, Ws):
    N, Cin, _ = x.shape
    Hp, Wp = H + 2, W + 2
    x4 = x.reshape(N, Cin, Hp, Wp)
    taps = jnp.concatenate([x4[:, :, r:r + H, c:c + W]
                            for r in range(2) for c in range(2)], axis=1)
    taps = taps.reshape(N, 4 * Cin, H * W)
    out = _mm_call(taps, w, b)
    Cout = w.shape[0] // 4
    out = out.reshape(N, 2, 2, Cout, H, W)
    out = jnp.transpose(out, (0, 3, 4, 1, 5, 2)).reshape(N, Cout, 2 * H, 2 * W)
    off_h = (2 * H + 1 - Hs) // 2
    off_w = (2 * W + 1 - Ws) // 2
    out = out[:, :, off_h:off_h + Hs, off_w:off_w + Ws]
    return _pad_flat(out)


def kernel(x, first_conv_w, first_conv_b, down0_0_scale, down0_0_shift,
           down0_0_w, down0_0_b, down0_1_scale, down0_1_shift, down0_1_w,
           down0_1_b, down1_0_scale, down1_0_shift, down1_0_w, down1_0_b,
           down1_1_scale, down1_1_shift, down1_1_w, down1_1_b, td0_scale,
           td0_shift, td0_w, td0_b, td1_scale, td1_shift, td1_w, td1_b,
           bn0_scale, bn0_shift, bn0_w, bn0_b, bn1_scale, bn1_shift, bn1_w,
           bn1_b, tu0_w, tu0_b, tu1_w, tu1_b, up0_0_scale, up0_0_shift,
           up0_0_w, up0_0_b, up0_1_scale, up0_1_shift, up0_1_w, up0_1_b,
           up1_0_scale, up1_0_shift, up1_0_w, up1_0_b, up1_1_scale,
           up1_1_shift, up1_1_w, up1_1_b, last_conv_w, last_conv_b):
    N, _, H, W = x.shape

    def stk(w):
        g, cin = w.shape[1], w.shape[2]
        return jnp.transpose(w.reshape(3, 3, g, cin),
                             (1, 0, 2, 3)).reshape(3, 3 * g, cin)
    C1 = first_conv_w.shape[1]
    fcw27 = jnp.transpose(first_conv_w, (1, 0, 2)).reshape(C1, 27)

    xp = _pad_flat(x)
    m0 = _ring_mask(H, W)

    a, d0 = _dense_block(
        [xp], m0,
        [(down0_0_scale, down0_0_shift, stk(down0_0_w), down0_0_b),
         (down0_1_scale, down0_1_shift, stk(down0_1_w), down0_1_b)],
        Wp=W + 2, first_conv=(fcw27, first_conv_b),
        emit_passthrough=True,
        td=(td0_scale, td0_shift, td0_w, td0_b))

    b1, H1, W1 = _pool2(d0, H, W)
    m1 = _ring_mask(H1, W1)

    c_new, d1 = _dense_block(
        [b1], m1,
        [(down1_0_scale, down1_0_shift, stk(down1_0_w), down1_0_b),
         (down1_1_scale, down1_1_shift, stk(down1_1_w), down1_1_b)],
        Wp=W1 + 2, emit_passthrough=False,
        td=(td1_scale, td1_shift, td1_w, td1_b))

    d, H2, W2 = _pool2(d1, H1, W1)
    m2 = _ring_mask(H2, W2)

    e = _dense_block(
        [d], m2,
        [(bn0_scale, bn0_shift, stk(bn0_w), bn0_b),
         (bn1_scale, bn1_shift, stk(bn1_w), bn1_b)],
        Wp=W2 + 2, emit_passthrough=False)

    u0 = _transition_up(e, tu0_w, tu0_b, H2, W2, H1, W1)
    f = _dense_block(
        [u0, b1, c_new], m1,
        [(up0_0_scale, up0_0_shift, stk(up0_0_w), up0_0_b),
         (up0_1_scale, up0_1_shift, stk(up0_1_w), up0_1_b)],
        Wp=W1 + 2, emit_passthrough=False)

    u1 = _transition_up(f, tu1_w, tu1_b, H1, W1, H, W)
    out = _dense_block(
        [u1, a], m0,
        [(up1_0_scale, up1_0_shift, stk(up1_0_w), up1_0_b),
         (up1_1_scale, up1_1_shift, stk(up1_1_w), up1_1_b)],
        Wp=W + 2, final_mm=(last_conv_w, last_conv_b))

    out = out.reshape(N, 1, H + 2, W + 2)[:, :, 1:H + 1, 1:W + 1]
    return out
